# baseline (speedup 1.0000x reference)
import functools

import jax
import jax.numpy as jnp
from jax import lax
from jax.experimental import pallas as pl
from jax.experimental.pallas import tpu as pltpu
from jax.experimental.pallas import tpu_sc as plsc

N = 10000
E = 320000
D = 128
DE = 16

NC = 2
NS = 16
NW = NC * NS
EPW = E // NW
C = 80
NCHUNK = EPW // C

_MESH1 = plsc.VectorSubcoreMesh(core_axis_name="c", subcore_axis_name="s",
                                num_cores=1)
E2 = E // 2
EPW2 = E2 // NS
NCHUNK2 = EPW2 // C


def _tc_a_body(x_ref, w1_ref, ad_ref, xw_ref, nl_ref):
    xb = x_ref[...]
    xw_ref[...] = jnp.dot(xb, w1_ref[...], preferred_element_type=jnp.float32)
    nl_ref[...] = jnp.sum(xb * ad_ref[...], axis=1, keepdims=True)


def _tc_a(x, w1, ad_row):
    blk = 400
    return pl.pallas_call(
        _tc_a_body,
        grid=(N // blk,),
        in_specs=[
            pl.BlockSpec((blk, D), lambda i: (i, 0)),
            pl.BlockSpec((D, D), lambda i: (0, 0)),
            pl.BlockSpec((1, D), lambda i: (0, 0)),
        ],
        out_specs=[
            pl.BlockSpec((blk, D), lambda i: (i, 0)),
            pl.BlockSpec((blk, 1), lambda i: (i, 0)),
        ],
        out_shape=[
            jax.ShapeDtypeStruct((N, D), jnp.float32),
            jax.ShapeDtypeStruct((N, 1), jnp.float32),
        ],
    )(x, w1, ad_row)


def _sc_gather_body(xw_hbm, nl_hbm, src_hbm, dst_hbm, g_hbm, nld_hbm,
                    idx_v, didx_v, rows_v, nld_v, sem):
    sid = lax.axis_index("s")
    base0 = sid * EPW2

    def chunk(k, carry):
        base = base0 + k * C
        pltpu.sync_copy(src_hbm.at[pl.ds(base, C)], idx_v)
        pltpu.sync_copy(dst_hbm.at[pl.ds(base, C)], didx_v)
        pltpu.async_copy(xw_hbm.at[idx_v], rows_v, sem).wait()
        pltpu.sync_copy(rows_v, g_hbm.at[pl.ds(base, C)])
        pltpu.async_copy(nl_hbm.at[didx_v], nld_v, sem).wait()
        pltpu.sync_copy(nld_v, nld_hbm.at[pl.ds(base, C)])
        return carry

    lax.fori_loop(0, NCHUNK2, chunk, 0)


def _sc_gather(xw, nl, src, dst):
    return pl.kernel(
        _sc_gather_body,
        out_type=[
            jax.ShapeDtypeStruct((E2, D), jnp.float32),
            jax.ShapeDtypeStruct((E2,), jnp.float32),
        ],
        mesh=_MESH1,
        scratch_types=[
            pltpu.VMEM((C,), jnp.int32),
            pltpu.VMEM((C,), jnp.int32),
            pltpu.VMEM((C, D), jnp.float32),
            pltpu.VMEM((C,), jnp.float32),
            pltpu.SemaphoreType.DMA,
        ],
    )(xw, nl, src, dst)


EB = 2000
GB = E2 // EB


def _tc_b1_body(g_ref, ea_ref, nld_ref, w2_ref, ae_ref, logit_ref, m_ref):
    i = pl.program_id(0)
    e = jnp.maximum(
        g_ref[...] + jnp.dot(ea_ref[...], w2_ref[...],
                             preferred_element_type=jnp.float32), 0.0)
    s = jnp.dot(e, ae_ref[...], preferred_element_type=jnp.float32) + nld_ref[...]
    l = jnp.where(s >= 0, s, 0.2 * s)
    logit_ref[...] = l

    @pl.when(i == 0)
    def _():
        m_ref[...] = jnp.full((1, 1), -jnp.inf, jnp.float32)

    m_ref[...] = jnp.maximum(m_ref[...], jnp.max(l))


def _tc_b1(g, ea, nld2, w2, ae):
    return pl.pallas_call(
        _tc_b1_body,
        grid=(GB,),
        in_specs=[
            pl.BlockSpec((EB, D), lambda i: (i, 0)),
            pl.BlockSpec((EB, DE), lambda i: (i, 0)),
            pl.BlockSpec((EB, 1), lambda i: (i, 0)),
            pl.BlockSpec((DE, D), lambda i: (0, 0)),
            pl.BlockSpec((D, 1), lambda i: (0, 0)),
        ],
        out_specs=[
            pl.BlockSpec((EB, 1), lambda i: (i, 0)),
            pl.BlockSpec((1, 1), lambda i: (0, 0)),
        ],
        out_shape=[
            jax.ShapeDtypeStruct((E2, 1), jnp.float32),
            jax.ShapeDtypeStruct((1, 1), jnp.float32),
        ],
    )(g, ea, nld2, w2, ae)


def _tc_b2_body(g_ref, ea_ref, logit_ref, m_ref, w2_ref, we_ref, ex_ref):
    e = jnp.maximum(
        g_ref[...] + jnp.dot(ea_ref[...], w2_ref[...],
                             preferred_element_type=jnp.float32), 0.0)
    ex = jnp.exp(logit_ref[...] - m_ref[...])
    ex_ref[...] = ex
    we_ref[...] = e * ex


def _tc_b2(g, ea, logit2, m, w2):
    return pl.pallas_call(
        _tc_b2_body,
        grid=(GB,),
        in_specs=[
            pl.BlockSpec((EB, D), lambda i: (i, 0)),
            pl.BlockSpec((EB, DE), lambda i: (i, 0)),
            pl.BlockSpec((EB, 1), lambda i: (i, 0)),
            pl.BlockSpec((1, 1), lambda i: (0, 0)),
            pl.BlockSpec((DE, D), lambda i: (0, 0)),
        ],
        out_specs=[
            pl.BlockSpec((EB, D), lambda i: (i, 0)),
            pl.BlockSpec((EB, 1), lambda i: (i, 0)),
        ],
        out_shape=[
            jax.ShapeDtypeStruct((E2, D), jnp.float32),
            jax.ShapeDtypeStruct((E2, 1), jnp.float32),
        ],
    )(g, ea, logit2, m, w2)


ZR = 624


def _sc_scatter_body(we_hbm, ex_hbm, dst_hbm, u_hbm, d_hbm,
                     wv, exv, didx_v, zb, u_sh, den_sh):
    sid = lax.axis_index("s")
    base0 = sid * EPW2

    zero16 = jnp.zeros((16,), jnp.float32)

    def z1(r, carry):
        for j in range(D // 16):
            zb[r, pl.ds(16 * j, 16)] = zero16
        return carry

    lax.fori_loop(0, 16, z1, 0)

    def zcopy(p, carry):
        pltpu.sync_copy(zb, u_sh.at[pl.ds(sid * ZR + p * 16, 16)])
        pltpu.sync_copy(zb.at[0, pl.ds(0, 16)],
                        den_sh.at[pl.ds(sid * ZR + p * 16, 16)])
        return carry

    lax.fori_loop(0, ZR // 16, zcopy, 0)

    @pl.when(sid == NS - 1)
    def _():
        pltpu.sync_copy(zb, u_sh.at[pl.ds(NS * ZR, N - NS * ZR)])
        pltpu.sync_copy(zb.at[0, pl.ds(0, 16)],
                        den_sh.at[pl.ds(NS * ZR, N - NS * ZR)])

    plsc.subcore_barrier()

    def chunk(k, carry):
        base = base0 + k * C
        pltpu.sync_copy(dst_hbm.at[pl.ds(base, C)], didx_v)
        pltpu.sync_copy(we_hbm.at[pl.ds(base, C)], wv)
        pltpu.sync_copy(wv, u_sh.at[didx_v], add=True)
        pltpu.sync_copy(ex_hbm.at[pl.ds(base, C)], exv)
        pltpu.sync_copy(exv, den_sh.at[didx_v], add=True)
        return carry

    lax.fori_loop(0, NCHUNK2, chunk, 0)
    plsc.subcore_barrier()

    pltpu.sync_copy(u_sh.at[pl.ds(sid * ZR, ZR)],
                    u_hbm.at[pl.ds(sid * ZR, ZR)])

    @pl.when(sid == NS - 1)
    def _():
        pltpu.sync_copy(u_sh.at[pl.ds(NS * ZR, N - NS * ZR)],
                        u_hbm.at[pl.ds(NS * ZR, N - NS * ZR)])

    @pl.when(sid == 0)
    def _():
        pltpu.sync_copy(den_sh, d_hbm)


def _sc_scatter(we, ex, dst):
    return pl.kernel(
        _sc_scatter_body,
        out_type=[
            jax.ShapeDtypeStruct((N, D), jnp.float32),
            jax.ShapeDtypeStruct((N,), jnp.float32),
        ],
        mesh=_MESH1,
        scratch_types=[
            pltpu.VMEM((C, D), jnp.float32),
            pltpu.VMEM((C,), jnp.float32),
            pltpu.VMEM((C,), jnp.int32),
            pltpu.VMEM((16, D), jnp.float32),
            pltpu.VMEM_SHARED((N, D), jnp.float32),
            pltpu.VMEM_SHARED((N,), jnp.float32),
        ],
    )(we, ex, dst)


def _tc_c_body(u0_ref, u1_ref, d0_ref, d1_ref, x_ref,
               wz1_ref, wz2_ref, wr1_ref, wr2_ref, wh1_ref, wh2_ref,
               bz_ref, br_ref, bh_ref, out_ref):
    den = d0_ref[...] + d1_ref[...] + 1e-9
    agg = (u0_ref[...] + u1_ref[...]) / den
    xb = x_ref[...]

    def mm(a, b):
        return jnp.dot(a, b, preferred_element_type=jnp.float32)

    z = jax.nn.sigmoid(mm(agg, wz1_ref[...]) + mm(xb, wz2_ref[...]) + bz_ref[...])
    r = jax.nn.sigmoid(mm(agg, wr1_ref[...]) + mm(xb, wr2_ref[...]) + br_ref[...])
    h = jnp.tanh(mm(agg, wh1_ref[...]) + mm(r * xb, wh2_ref[...]) + bh_ref[...])
    out_ref[...] = (1.0 - z) * xb + z * h


def _tc_c(u0, u1, d0, d1, x, wz1, wz2, wr1, wr2, wh1, wh2, bz, br, bh):
    blk = 400
    nspec = pl.BlockSpec((blk, D), lambda i: (i, 0))
    dspec = pl.BlockSpec((blk, 1), lambda i: (i, 0))
    wspec = pl.BlockSpec((D, D), lambda i: (0, 0))
    bspec = pl.BlockSpec((1, D), lambda i: (0, 0))
    return pl.pallas_call(
        _tc_c_body,
        grid=(N // blk,),
        in_specs=[nspec, nspec, dspec, dspec, nspec,
                  wspec, wspec, wspec, wspec, wspec, wspec,
                  bspec, bspec, bspec],
        out_specs=nspec,
        out_shape=jax.ShapeDtypeStruct((N, D), jnp.float32),
    )(u0, u1, d0, d1, x, wz1, wz2, wr1, wr2, wh1, wh2, bz, br, bh)


def kernel(x, edge_attr, W_edge, a_att, W_z, b_z, W_r, b_r, W_h, b_h, edge_index):
    src = edge_index[0].astype(jnp.int32)
    dst = edge_index[1].astype(jnp.int32)
    w1 = W_edge[:D]
    w2 = W_edge[D:]
    ad_row = a_att[:D, 0].reshape(1, D)
    ae = a_att[D:]

    xw, nl2 = _tc_a(x, w1, ad_row)
    nl = nl2.reshape(N)

    g0, nld0 = _sc_gather(xw, nl, src[:E2], dst[:E2])
    g1, nld1 = _sc_gather(xw, nl, src[E2:], dst[E2:])
    ea0, ea1 = edge_attr[:E2], edge_attr[E2:]

    logit0, m0 = _tc_b1(g0, ea0, nld0.reshape(E2, 1), w2, ae)
    logit1, m1 = _tc_b1(g1, ea1, nld1.reshape(E2, 1), w2, ae)
    m = jnp.maximum(m0, m1)

    we0, ex0 = _tc_b2(g0, ea0, logit0, m, w2)
    we1, ex1 = _tc_b2(g1, ea1, logit1, m, w2)

    u0, d0 = _sc_scatter(we0, ex0.reshape(E2), dst[:E2])
    u1, d1 = _sc_scatter(we1, ex1.reshape(E2), dst[E2:])

    out = _tc_c(u0, u1, d0.reshape(N, 1), d1.reshape(N, 1), x,
                W_z[:D], W_z[D:], W_r[:D], W_r[D:], W_h[:D], W_h[D:],
                b_z.reshape(1, D), b_r.reshape(1, D), b_h.reshape(1, D))
    return out


# speedup vs baseline: 1.5489x; 1.5489x over previous
import functools

import jax
import jax.numpy as jnp
from jax import lax
from jax.experimental import pallas as pl
from jax.experimental.pallas import tpu as pltpu
from jax.experimental.pallas import tpu_sc as plsc

N = 10000
E = 320000
D = 128
DE = 16

NC = 2
NS = 16
NW = NC * NS
EPW = E // NW
C = 128
NCHUNK = EPW // C
CT = EPW - NCHUNK * C

_MESH = plsc.VectorSubcoreMesh(core_axis_name="c", subcore_axis_name="s")


def _tc_a_body(x_ref, w1_ref, ad_ref, xw_ref, nl_ref):
    xb = x_ref[...]
    xw_ref[...] = jnp.dot(xb, w1_ref[...], preferred_element_type=jnp.float32)
    nl_ref[...] = jnp.sum(xb * ad_ref[...], axis=1, keepdims=True)


def _tc_a(x, w1, ad_row):
    blk = 400
    return pl.pallas_call(
        _tc_a_body,
        grid=(N // blk,),
        in_specs=[
            pl.BlockSpec((blk, D), lambda i: (i, 0)),
            pl.BlockSpec((D, D), lambda i: (0, 0)),
            pl.BlockSpec((1, D), lambda i: (0, 0)),
        ],
        out_specs=[
            pl.BlockSpec((blk, D), lambda i: (i, 0)),
            pl.BlockSpec((blk, 1), lambda i: (i, 0)),
        ],
        out_shape=[
            jax.ShapeDtypeStruct((N, D), jnp.float32),
            jax.ShapeDtypeStruct((N, 1), jnp.float32),
        ],
    )(x, w1, ad_row)


def _sc_gather_body(xw_hbm, nl_hbm, src_hbm, dst_hbm, g_hbm, nld_hbm,
                    idx_v, didx_v, rows_v, nld_v, idxt_v, didxt_v,
                    rowst_v, nldt_v, sem):
    wid = lax.axis_index("s") * NC + lax.axis_index("c")
    base0 = wid * EPW

    def chunk(k, carry):
        base = base0 + k * C
        pltpu.sync_copy(src_hbm.at[pl.ds(base, C)], idx_v)
        pltpu.sync_copy(dst_hbm.at[pl.ds(base, C)], didx_v)
        pltpu.async_copy(xw_hbm.at[idx_v], rows_v, sem).wait()
        pltpu.sync_copy(rows_v, g_hbm.at[pl.ds(base, C)])
        pltpu.async_copy(nl_hbm.at[didx_v], nld_v, sem).wait()
        pltpu.sync_copy(nld_v, nld_hbm.at[pl.ds(base, C)])
        return carry

    lax.fori_loop(0, NCHUNK, chunk, 0)

    base = base0 + NCHUNK * C
    pltpu.sync_copy(src_hbm.at[pl.ds(base, CT)], idxt_v)
    pltpu.sync_copy(dst_hbm.at[pl.ds(base, CT)], didxt_v)
    pltpu.async_copy(xw_hbm.at[idxt_v], rowst_v, sem).wait()
    pltpu.sync_copy(rowst_v, g_hbm.at[pl.ds(base, CT)])
    pltpu.async_copy(nl_hbm.at[didxt_v], nldt_v, sem).wait()
    pltpu.sync_copy(nldt_v, nld_hbm.at[pl.ds(base, CT)])


def _sc_gather(xw, nl, src, dst):
    return pl.kernel(
        _sc_gather_body,
        out_type=[
            jax.ShapeDtypeStruct((E, D), jnp.float32),
            jax.ShapeDtypeStruct((E,), jnp.float32),
        ],
        mesh=_MESH,
        scratch_types=[
            pltpu.VMEM((C,), jnp.int32),
            pltpu.VMEM((C,), jnp.int32),
            pltpu.VMEM((C, D), jnp.float32),
            pltpu.VMEM((C,), jnp.float32),
            pltpu.VMEM((CT,), jnp.int32),
            pltpu.VMEM((CT,), jnp.int32),
            pltpu.VMEM((CT, D), jnp.float32),
            pltpu.VMEM((CT,), jnp.float32),
            pltpu.SemaphoreType.DMA,
        ],
    )(xw, nl, src, dst)


EB = 2560
GB = E // EB


def _tc_b1_body(g_ref, ea_ref, nld_ref, w2_ref, ae_ref, logit_ref, m_ref):
    i = pl.program_id(0)
    e = jnp.maximum(
        g_ref[...] + jnp.dot(ea_ref[...], w2_ref[...],
                             preferred_element_type=jnp.float32), 0.0)
    s = jnp.dot(e, ae_ref[...], preferred_element_type=jnp.float32) + nld_ref[...]
    l = jnp.where(s >= 0, s, 0.2 * s)
    logit_ref[...] = l

    @pl.when(i == 0)
    def _():
        m_ref[...] = jnp.full((1, 1), -jnp.inf, jnp.float32)

    m_ref[...] = jnp.maximum(m_ref[...], jnp.max(l))


def _tc_b1(g, ea, nld2, w2, ae):
    return pl.pallas_call(
        _tc_b1_body,
        grid=(GB,),
        in_specs=[
            pl.BlockSpec((EB, D), lambda i: (i, 0)),
            pl.BlockSpec((EB, DE), lambda i: (i, 0)),
            pl.BlockSpec((EB, 1), lambda i: (i, 0)),
            pl.BlockSpec((DE, D), lambda i: (0, 0)),
            pl.BlockSpec((D, 1), lambda i: (0, 0)),
        ],
        out_specs=[
            pl.BlockSpec((EB, 1), lambda i: (i, 0)),
            pl.BlockSpec((1, 1), lambda i: (0, 0)),
        ],
        out_shape=[
            jax.ShapeDtypeStruct((E, 1), jnp.float32),
            jax.ShapeDtypeStruct((1, 1), jnp.float32),
        ],
    )(g, ea, nld2, w2, ae)


def _tc_b2_body(g_ref, ea_ref, logit_ref, m_ref, w2_ref, we_ref, ex_ref):
    e = jnp.maximum(
        g_ref[...] + jnp.dot(ea_ref[...], w2_ref[...],
                             preferred_element_type=jnp.float32), 0.0)
    ex = jnp.exp(logit_ref[...] - m_ref[...])
    ex_ref[...] = ex
    we_ref[...] = e * ex


def _tc_b2(g, ea, logit2, m, w2):
    return pl.pallas_call(
        _tc_b2_body,
        grid=(GB,),
        in_specs=[
            pl.BlockSpec((EB, D), lambda i: (i, 0)),
            pl.BlockSpec((EB, DE), lambda i: (i, 0)),
            pl.BlockSpec((EB, 1), lambda i: (i, 0)),
            pl.BlockSpec((1, 1), lambda i: (0, 0)),
            pl.BlockSpec((DE, D), lambda i: (0, 0)),
        ],
        out_specs=[
            pl.BlockSpec((EB, D), lambda i: (i, 0)),
            pl.BlockSpec((EB, 1), lambda i: (i, 0)),
        ],
        out_shape=[
            jax.ShapeDtypeStruct((E, D), jnp.float32),
            jax.ShapeDtypeStruct((E, 1), jnp.float32),
        ],
    )(g, ea, logit2, m, w2)


ZR = 624


def _sc_scatter_body(we_hbm, ex_hbm, dst_hbm, u0_hbm, u1_hbm, d0_hbm, d1_hbm,
                     wv, exv, didx_v, wvt, exvt, didxt_v, zb, u_sh, den_sh):
    cid = lax.axis_index("c")
    sid = lax.axis_index("s")
    wid = sid * NC + cid
    base0 = wid * EPW

    zero16 = jnp.zeros((16,), jnp.float32)

    def z1(r, carry):
        for j in range(D // 16):
            zb[r, pl.ds(16 * j, 16)] = zero16
        return carry

    lax.fori_loop(0, 16, z1, 0)

    def zcopy(p, carry):
        pltpu.sync_copy(zb, u_sh.at[pl.ds(sid * ZR + p * 16, 16)])
        pltpu.sync_copy(zb.at[0, pl.ds(0, 16)],
                        den_sh.at[pl.ds(sid * ZR + p * 16, 16)])
        return carry

    lax.fori_loop(0, ZR // 16, zcopy, 0)

    @pl.when(sid == NS - 1)
    def _():
        pltpu.sync_copy(zb, u_sh.at[pl.ds(NS * ZR, N - NS * ZR)])
        pltpu.sync_copy(zb.at[0, pl.ds(0, 16)],
                        den_sh.at[pl.ds(NS * ZR, N - NS * ZR)])

    plsc.subcore_barrier()

    def chunk(k, carry):
        base = base0 + k * C
        pltpu.sync_copy(dst_hbm.at[pl.ds(base, C)], didx_v)
        pltpu.sync_copy(we_hbm.at[pl.ds(base, C)], wv)
        pltpu.sync_copy(wv, u_sh.at[didx_v], add=True)
        pltpu.sync_copy(ex_hbm.at[pl.ds(base, C)], exv)
        pltpu.sync_copy(exv, den_sh.at[didx_v], add=True)
        return carry

    lax.fori_loop(0, NCHUNK, chunk, 0)

    base = base0 + NCHUNK * C
    pltpu.sync_copy(dst_hbm.at[pl.ds(base, CT)], didxt_v)
    pltpu.sync_copy(we_hbm.at[pl.ds(base, CT)], wvt)
    pltpu.sync_copy(wvt, u_sh.at[didxt_v], add=True)
    pltpu.sync_copy(ex_hbm.at[pl.ds(base, CT)], exvt)
    pltpu.sync_copy(exvt, den_sh.at[didxt_v], add=True)
    plsc.subcore_barrier()

    def dump(u_hbm, d_hbm):
        pltpu.sync_copy(u_sh.at[pl.ds(sid * ZR, ZR)],
                        u_hbm.at[pl.ds(sid * ZR, ZR)])

        @pl.when(sid == NS - 1)
        def _():
            pltpu.sync_copy(u_sh.at[pl.ds(NS * ZR, N - NS * ZR)],
                            u_hbm.at[pl.ds(NS * ZR, N - NS * ZR)])

        @pl.when(sid == 0)
        def _():
            pltpu.sync_copy(den_sh, d_hbm)

    @pl.when(cid == 0)
    def _():
        dump(u0_hbm, d0_hbm)

    @pl.when(cid == 1)
    def _():
        dump(u1_hbm, d1_hbm)


def _sc_scatter(we, ex, dst):
    return pl.kernel(
        _sc_scatter_body,
        out_type=[
            jax.ShapeDtypeStruct((N, D), jnp.float32),
            jax.ShapeDtypeStruct((N, D), jnp.float32),
            jax.ShapeDtypeStruct((N,), jnp.float32),
            jax.ShapeDtypeStruct((N,), jnp.float32),
        ],
        mesh=_MESH,
        scratch_types=[
            pltpu.VMEM((C, D), jnp.float32),
            pltpu.VMEM((C,), jnp.float32),
            pltpu.VMEM((C,), jnp.int32),
            pltpu.VMEM((CT, D), jnp.float32),
            pltpu.VMEM((CT,), jnp.float32),
            pltpu.VMEM((CT,), jnp.int32),
            pltpu.VMEM((16, D), jnp.float32),
            pltpu.VMEM_SHARED((N, D), jnp.float32),
            pltpu.VMEM_SHARED((N,), jnp.float32),
        ],
    )(we, ex, dst)


def _tc_c_body(u0_ref, u1_ref, d0_ref, d1_ref, x_ref,
               wz1_ref, wz2_ref, wr1_ref, wr2_ref, wh1_ref, wh2_ref,
               bz_ref, br_ref, bh_ref, out_ref):
    den = d0_ref[...] + d1_ref[...] + 1e-9
    agg = (u0_ref[...] + u1_ref[...]) / den
    xb = x_ref[...]

    def mm(a, b):
        return jnp.dot(a, b, preferred_element_type=jnp.float32)

    z = jax.nn.sigmoid(mm(agg, wz1_ref[...]) + mm(xb, wz2_ref[...]) + bz_ref[...])
    r = jax.nn.sigmoid(mm(agg, wr1_ref[...]) + mm(xb, wr2_ref[...]) + br_ref[...])
    h = jnp.tanh(mm(agg, wh1_ref[...]) + mm(r * xb, wh2_ref[...]) + bh_ref[...])
    out_ref[...] = (1.0 - z) * xb + z * h


def _tc_c(u0, u1, d0, d1, x, wz1, wz2, wr1, wr2, wh1, wh2, bz, br, bh):
    blk = 400
    nspec = pl.BlockSpec((blk, D), lambda i: (i, 0))
    dspec = pl.BlockSpec((blk, 1), lambda i: (i, 0))
    wspec = pl.BlockSpec((D, D), lambda i: (0, 0))
    bspec = pl.BlockSpec((1, D), lambda i: (0, 0))
    return pl.pallas_call(
        _tc_c_body,
        grid=(N // blk,),
        in_specs=[nspec, nspec, dspec, dspec, nspec,
                  wspec, wspec, wspec, wspec, wspec, wspec,
                  bspec, bspec, bspec],
        out_specs=nspec,
        out_shape=jax.ShapeDtypeStruct((N, D), jnp.float32),
    )(u0, u1, d0, d1, x, wz1, wz2, wr1, wr2, wh1, wh2, bz, br, bh)


def kernel(x, edge_attr, W_edge, a_att, W_z, b_z, W_r, b_r, W_h, b_h, edge_index):
    src = edge_index[0].astype(jnp.int32)
    dst = edge_index[1].astype(jnp.int32)
    w1 = W_edge[:D]
    w2 = W_edge[D:]
    ad_row = a_att[:D, 0].reshape(1, D)
    ae = a_att[D:]

    xw, nl2 = _tc_a(x, w1, ad_row)
    g, nld = _sc_gather(xw, nl2.reshape(N), src, dst)
    logit2, m = _tc_b1(g, edge_attr, nld.reshape(E, 1), w2, ae)
    we, ex2 = _tc_b2(g, edge_attr, logit2, m, w2)
    u0, u1, d0, d1 = _sc_scatter(we, ex2.reshape(E), dst)

    out = _tc_c(u0, u1, d0.reshape(N, 1), d1.reshape(N, 1), x,
                W_z[:D], W_z[D:], W_r[:D], W_r[D:], W_h[:D], W_h[D:],
                b_z.reshape(1, D), b_r.reshape(1, D), b_h.reshape(1, D))
    return out


# speedup vs baseline: 1.7256x; 1.1141x over previous
import functools

import jax
import jax.numpy as jnp
from jax import lax
from jax.experimental import pallas as pl
from jax.experimental.pallas import tpu as pltpu
from jax.experimental.pallas import tpu_sc as plsc

N = 10000
E = 320000
D = 128
DE = 16

NC = 2
NS = 16
NW = NC * NS
EPW = E // NW
C = 128
NCHUNK = EPW // C
CT = EPW - NCHUNK * C

_MESH = plsc.VectorSubcoreMesh(core_axis_name="c", subcore_axis_name="s")


def _tc_a_body(x_ref, w1_ref, ad_ref, xw_ref, nl_ref):
    xb = x_ref[...]
    xw_ref[...] = jnp.dot(xb, w1_ref[...], preferred_element_type=jnp.float32)
    nl_ref[...] = jnp.sum(xb * ad_ref[...], axis=1, keepdims=True)


def _tc_a(x, w1, ad_row):
    blk = 400
    return pl.pallas_call(
        _tc_a_body,
        grid=(N // blk,),
        in_specs=[
            pl.BlockSpec((blk, D), lambda i: (i, 0)),
            pl.BlockSpec((D, D), lambda i: (0, 0)),
            pl.BlockSpec((1, D), lambda i: (0, 0)),
        ],
        out_specs=[
            pl.BlockSpec((blk, D), lambda i: (i, 0)),
            pl.BlockSpec((blk, 1), lambda i: (i, 0)),
        ],
        out_shape=[
            jax.ShapeDtypeStruct((N, D), jnp.float32),
            jax.ShapeDtypeStruct((N, 1), jnp.float32),
        ],
    )(x, w1, ad_row)


def _sc_gather_body(xw_hbm, nl_hbm, src_hbm, dst_hbm, g_hbm, nld_hbm,
                    idx_v, didx_v, rows_v, nld_v, idx2_v, didx2_v,
                    rows2_v, nld2_v, idxt_v, didxt_v, rowst_v, nldt_v,
                    sema, semb, sem):
    wid = lax.axis_index("s") * NC + lax.axis_index("c")
    base0 = wid * EPW

    def issue(k, iv, div, rv, nv, sm):
        base = base0 + k * C
        pltpu.sync_copy(src_hbm.at[pl.ds(base, C)], iv)
        pltpu.sync_copy(dst_hbm.at[pl.ds(base, C)], div)
        pltpu.async_copy(xw_hbm.at[iv], rv, sm)
        pltpu.async_copy(nl_hbm.at[div], nv, sm)

    def drain_wb(k, iv, div, rv, nv, sm):
        base = base0 + k * C
        pltpu.make_async_copy(xw_hbm.at[iv], rv, sm).wait()
        pltpu.make_async_copy(nl_hbm.at[div], nv, sm).wait()
        pltpu.sync_copy(rv, g_hbm.at[pl.ds(base, C)])
        pltpu.sync_copy(nv, nld_hbm.at[pl.ds(base, C)])

    issue(0, idx_v, didx_v, rows_v, nld_v, sema)

    def pair(k2, carry):
        c0 = 2 * k2
        issue(c0 + 1, idx2_v, didx2_v, rows2_v, nld2_v, semb)
        drain_wb(c0, idx_v, didx_v, rows_v, nld_v, sema)

        @pl.when(k2 < NCHUNK // 2 - 1)
        def _():
            issue(c0 + 2, idx_v, didx_v, rows_v, nld_v, sema)

        drain_wb(c0 + 1, idx2_v, didx2_v, rows2_v, nld2_v, semb)
        return carry

    lax.fori_loop(0, NCHUNK // 2, pair, 0)

    base = base0 + NCHUNK * C
    pltpu.sync_copy(src_hbm.at[pl.ds(base, CT)], idxt_v)
    pltpu.sync_copy(dst_hbm.at[pl.ds(base, CT)], didxt_v)
    pltpu.async_copy(xw_hbm.at[idxt_v], rowst_v, sem).wait()
    pltpu.sync_copy(rowst_v, g_hbm.at[pl.ds(base, CT)])
    pltpu.async_copy(nl_hbm.at[didxt_v], nldt_v, sem).wait()
    pltpu.sync_copy(nldt_v, nld_hbm.at[pl.ds(base, CT)])


def _sc_gather(xw, nl, src, dst):
    return pl.kernel(
        _sc_gather_body,
        out_type=[
            jax.ShapeDtypeStruct((E, D), jnp.float32),
            jax.ShapeDtypeStruct((E,), jnp.float32),
        ],
        mesh=_MESH,
        scratch_types=[
            pltpu.VMEM((C,), jnp.int32),
            pltpu.VMEM((C,), jnp.int32),
            pltpu.VMEM((C, D), jnp.float32),
            pltpu.VMEM((C,), jnp.float32),
            pltpu.VMEM((C,), jnp.int32),
            pltpu.VMEM((C,), jnp.int32),
            pltpu.VMEM((C, D), jnp.float32),
            pltpu.VMEM((C,), jnp.float32),
            pltpu.VMEM((CT,), jnp.int32),
            pltpu.VMEM((CT,), jnp.int32),
            pltpu.VMEM((CT, D), jnp.float32),
            pltpu.VMEM((CT,), jnp.float32),
            pltpu.SemaphoreType.DMA,
            pltpu.SemaphoreType.DMA,
            pltpu.SemaphoreType.DMA,
        ],
    )(xw, nl, src, dst)


EB = 2560
GB = E // EB


def _tc_b1_body(g_ref, ea_ref, nld_ref, w2_ref, ae_ref, logit_ref, m_ref):
    i = pl.program_id(0)
    e = jnp.maximum(
        g_ref[...] + jnp.dot(ea_ref[...], w2_ref[...],
                             preferred_element_type=jnp.float32), 0.0)
    s = jnp.dot(e, ae_ref[...], preferred_element_type=jnp.float32) + nld_ref[...]
    l = jnp.where(s >= 0, s, 0.2 * s)
    logit_ref[...] = l

    @pl.when(i == 0)
    def _():
        m_ref[...] = jnp.full((1, 1), -jnp.inf, jnp.float32)

    m_ref[...] = jnp.maximum(m_ref[...], jnp.max(l))


def _tc_b1(g, ea, nld2, w2, ae):
    return pl.pallas_call(
        _tc_b1_body,
        grid=(GB,),
        in_specs=[
            pl.BlockSpec((EB, D), lambda i: (i, 0)),
            pl.BlockSpec((EB, DE), lambda i: (i, 0)),
            pl.BlockSpec((EB, 1), lambda i: (i, 0)),
            pl.BlockSpec((DE, D), lambda i: (0, 0)),
            pl.BlockSpec((D, 1), lambda i: (0, 0)),
        ],
        out_specs=[
            pl.BlockSpec((EB, 1), lambda i: (i, 0)),
            pl.BlockSpec((1, 1), lambda i: (0, 0)),
        ],
        out_shape=[
            jax.ShapeDtypeStruct((E, 1), jnp.float32),
            jax.ShapeDtypeStruct((1, 1), jnp.float32),
        ],
    )(g, ea, nld2, w2, ae)


def _tc_b2_body(g_ref, ea_ref, logit_ref, m_ref, w2_ref, we_ref, ex_ref):
    e = jnp.maximum(
        g_ref[...] + jnp.dot(ea_ref[...], w2_ref[...],
                             preferred_element_type=jnp.float32), 0.0)
    ex = jnp.exp(logit_ref[...] - m_ref[...])
    ex_ref[...] = ex
    we_ref[...] = e * ex


def _tc_b2(g, ea, logit2, m, w2):
    return pl.pallas_call(
        _tc_b2_body,
        grid=(GB,),
        in_specs=[
            pl.BlockSpec((EB, D), lambda i: (i, 0)),
            pl.BlockSpec((EB, DE), lambda i: (i, 0)),
            pl.BlockSpec((EB, 1), lambda i: (i, 0)),
            pl.BlockSpec((1, 1), lambda i: (0, 0)),
            pl.BlockSpec((DE, D), lambda i: (0, 0)),
        ],
        out_specs=[
            pl.BlockSpec((EB, D), lambda i: (i, 0)),
            pl.BlockSpec((EB, 1), lambda i: (i, 0)),
        ],
        out_shape=[
            jax.ShapeDtypeStruct((E, D), jnp.float32),
            jax.ShapeDtypeStruct((E, 1), jnp.float32),
        ],
    )(g, ea, logit2, m, w2)


ZR = 624


def _sc_scatter_body(we_hbm, ex_hbm, dst_hbm, u0_hbm, u1_hbm, d0_hbm, d1_hbm,
                     wv, exv, didx_v, wvt, exvt, didxt_v, zb, u_sh, den_sh):
    cid = lax.axis_index("c")
    sid = lax.axis_index("s")
    wid = sid * NC + cid
    base0 = wid * EPW

    zero16 = jnp.zeros((16,), jnp.float32)

    def z1(r, carry):
        for j in range(D // 16):
            zb[r, pl.ds(16 * j, 16)] = zero16
        return carry

    lax.fori_loop(0, 16, z1, 0)

    def zcopy(p, carry):
        pltpu.sync_copy(zb, u_sh.at[pl.ds(sid * ZR + p * 16, 16)])
        pltpu.sync_copy(zb.at[0, pl.ds(0, 16)],
                        den_sh.at[pl.ds(sid * ZR + p * 16, 16)])
        return carry

    lax.fori_loop(0, ZR // 16, zcopy, 0)

    @pl.when(sid == NS - 1)
    def _():
        pltpu.sync_copy(zb, u_sh.at[pl.ds(NS * ZR, N - NS * ZR)])
        pltpu.sync_copy(zb.at[0, pl.ds(0, 16)],
                        den_sh.at[pl.ds(NS * ZR, N - NS * ZR)])

    plsc.subcore_barrier()

    def chunk(k, carry):
        base = base0 + k * C
        pltpu.sync_copy(dst_hbm.at[pl.ds(base, C)], didx_v)
        pltpu.sync_copy(we_hbm.at[pl.ds(base, C)], wv)
        pltpu.sync_copy(wv, u_sh.at[didx_v], add=True)
        pltpu.sync_copy(ex_hbm.at[pl.ds(base, C)], exv)
        pltpu.sync_copy(exv, den_sh.at[didx_v], add=True)
        return carry

    lax.fori_loop(0, NCHUNK, chunk, 0)

    base = base0 + NCHUNK * C
    pltpu.sync_copy(dst_hbm.at[pl.ds(base, CT)], didxt_v)
    pltpu.sync_copy(we_hbm.at[pl.ds(base, CT)], wvt)
    pltpu.sync_copy(wvt, u_sh.at[didxt_v], add=True)
    pltpu.sync_copy(ex_hbm.at[pl.ds(base, CT)], exvt)
    pltpu.sync_copy(exvt, den_sh.at[didxt_v], add=True)
    plsc.subcore_barrier()

    def dump(u_hbm, d_hbm):
        pltpu.sync_copy(u_sh.at[pl.ds(sid * ZR, ZR)],
                        u_hbm.at[pl.ds(sid * ZR, ZR)])

        @pl.when(sid == NS - 1)
        def _():
            pltpu.sync_copy(u_sh.at[pl.ds(NS * ZR, N - NS * ZR)],
                            u_hbm.at[pl.ds(NS * ZR, N - NS * ZR)])

        @pl.when(sid == 0)
        def _():
            pltpu.sync_copy(den_sh, d_hbm)

    @pl.when(cid == 0)
    def _():
        dump(u0_hbm, d0_hbm)

    @pl.when(cid == 1)
    def _():
        dump(u1_hbm, d1_hbm)


def _sc_scatter(we, ex, dst):
    return pl.kernel(
        _sc_scatter_body,
        out_type=[
            jax.ShapeDtypeStruct((N, D), jnp.float32),
            jax.ShapeDtypeStruct((N, D), jnp.float32),
            jax.ShapeDtypeStruct((N,), jnp.float32),
            jax.ShapeDtypeStruct((N,), jnp.float32),
        ],
        mesh=_MESH,
        scratch_types=[
            pltpu.VMEM((C, D), jnp.float32),
            pltpu.VMEM((C,), jnp.float32),
            pltpu.VMEM((C,), jnp.int32),
            pltpu.VMEM((CT, D), jnp.float32),
            pltpu.VMEM((CT,), jnp.float32),
            pltpu.VMEM((CT,), jnp.int32),
            pltpu.VMEM((16, D), jnp.float32),
            pltpu.VMEM_SHARED((N, D), jnp.float32),
            pltpu.VMEM_SHARED((N,), jnp.float32),
        ],
    )(we, ex, dst)


def _tc_c_body(u0_ref, u1_ref, d0_ref, d1_ref, x_ref,
               wz1_ref, wz2_ref, wr1_ref, wr2_ref, wh1_ref, wh2_ref,
               bz_ref, br_ref, bh_ref, out_ref):
    den = d0_ref[...] + d1_ref[...] + 1e-9
    agg = (u0_ref[...] + u1_ref[...]) / den
    xb = x_ref[...]

    def mm(a, b):
        return jnp.dot(a, b, preferred_element_type=jnp.float32)

    z = jax.nn.sigmoid(mm(agg, wz1_ref[...]) + mm(xb, wz2_ref[...]) + bz_ref[...])
    r = jax.nn.sigmoid(mm(agg, wr1_ref[...]) + mm(xb, wr2_ref[...]) + br_ref[...])
    h = jnp.tanh(mm(agg, wh1_ref[...]) + mm(r * xb, wh2_ref[...]) + bh_ref[...])
    out_ref[...] = (1.0 - z) * xb + z * h


def _tc_c(u0, u1, d0, d1, x, wz1, wz2, wr1, wr2, wh1, wh2, bz, br, bh):
    blk = 400
    nspec = pl.BlockSpec((blk, D), lambda i: (i, 0))
    dspec = pl.BlockSpec((blk, 1), lambda i: (i, 0))
    wspec = pl.BlockSpec((D, D), lambda i: (0, 0))
    bspec = pl.BlockSpec((1, D), lambda i: (0, 0))
    return pl.pallas_call(
        _tc_c_body,
        grid=(N // blk,),
        in_specs=[nspec, nspec, dspec, dspec, nspec,
                  wspec, wspec, wspec, wspec, wspec, wspec,
                  bspec, bspec, bspec],
        out_specs=nspec,
        out_shape=jax.ShapeDtypeStruct((N, D), jnp.float32),
    )(u0, u1, d0, d1, x, wz1, wz2, wr1, wr2, wh1, wh2, bz, br, bh)


def kernel(x, edge_attr, W_edge, a_att, W_z, b_z, W_r, b_r, W_h, b_h, edge_index):
    src = edge_index[0].astype(jnp.int32)
    dst = edge_index[1].astype(jnp.int32)
    w1 = W_edge[:D]
    w2 = W_edge[D:]
    ad_row = a_att[:D, 0].reshape(1, D)
    ae = a_att[D:]

    xw, nl2 = _tc_a(x, w1, ad_row)
    g, nld = _sc_gather(xw, nl2.reshape(N), src, dst)
    logit2, m = _tc_b1(g, edge_attr, nld.reshape(E, 1), w2, ae)
    we, ex2 = _tc_b2(g, edge_attr, logit2, m, w2)
    u0, u1, d0, d1 = _sc_scatter(we, ex2.reshape(E), dst)

    out = _tc_c(u0, u1, d0.reshape(N, 1), d1.reshape(N, 1), x,
                W_z[:D], W_z[D:], W_r[:D], W_r[D:], W_h[:D], W_h[D:],
                b_z.reshape(1, D), b_r.reshape(1, D), b_h.reshape(1, D))
    return out
